# baseline (speedup 1.0000x reference)
import functools

import jax
import jax.numpy as jnp
from jax import lax
from jax.experimental import pallas as pl
from jax.experimental.pallas import tpu as pltpu
from jax.experimental.pallas import tpu_sc as plsc

DEPTH = 4
N_ATOMS = 10000
N_BONDS = 320000
MAX_NB = 32
ATOM_FDIM = 128
BOND_FDIM = 144
HIDDEN = 64

NC, NS = 2, 16
NW = NC * NS
CH = 128
NB_PAD = 327680
NA_PAD = 10240
BOND_BLK = 4096
ATOM_BLK = 512


_A_PER_CH = CH // MAX_NB
_HV = HIDDEN // 16


def _sc_gather_sum(table, idx2d):
    D = table.shape[1]
    rows_per_w = NB_PAD // NW
    n_ch = rows_per_w // CH
    a_per_w = NA_PAD // NW
    K = 4
    n_sch = n_ch // K
    mesh = plsc.VectorSubcoreMesh(core_axis_name="c", subcore_axis_name="s")

    @functools.partial(
        pl.kernel,
        out_type=jax.ShapeDtypeStruct((NA_PAD, D), jnp.float32),
        mesh=mesh,
        compiler_params=pltpu.CompilerParams(use_tc_tiling_on_sc=False),
        scratch_types=[
            pltpu.VMEM((n_ch, CH), jnp.int32),
            pltpu.VMEM((4 * CH, D), jnp.float32),
            pltpu.VMEM((4 * CH, D), jnp.float32),
            pltpu.VMEM((a_per_w, D), jnp.float32),
            pltpu.SemaphoreType.DMA,
            pltpu.SemaphoreType.DMA,
        ],
    )
    def gsum_k(table_hbm, idx_hbm, out_hbm, idx_v, buf0, buf1, acc_v, sem0, sem1):
        wid = lax.axis_index("s") * NC + lax.axis_index("c")
        pltpu.sync_copy(idx_hbm.at[pl.ds(wid * n_ch, n_ch)], idx_v)

        def _fire(s, buf, sem):
            for b in range(K):
                pltpu.async_copy(table_hbm.at[idx_v.at[s * K + b]],
                                 buf.at[pl.ds(b * CH, CH)], sem)

        def _drain(buf, sem):
            pltpu.make_async_copy(table_hbm.at[pl.ds(0, K * CH)], buf, sem).wait()

        def _reduce(s, buf):
            def one_chunk(c, carry):
                for a in range(_A_PER_CH):
                    r0 = a * MAX_NB
                    for k in range(_HV):
                        acc = buf[c * CH + r0, pl.ds(k * 16, 16)]
                        for j in range(1, MAX_NB):
                            acc = acc + buf[c * CH + r0 + j, pl.ds(k * 16, 16)]
                        acc_v[(s * K + c) * _A_PER_CH + a, pl.ds(k * 16, 16)] = acc
                return carry
            lax.fori_loop(0, K, one_chunk, 0)

        _fire(0, buf0, sem0)

        def outer(g, carry):
            s0 = g * 2
            _fire(s0 + 1, buf1, sem1)
            _drain(buf0, sem0)
            _reduce(s0, buf0)

            @pl.when(s0 + 2 < n_sch)
            def _():
                _fire(s0 + 2, buf0, sem0)

            _drain(buf1, sem1)
            _reduce(s0 + 1, buf1)
            return carry

        lax.fori_loop(0, n_sch // 2, outer, 0)
        pltpu.sync_copy(acc_v, out_hbm.at[pl.ds(wid * a_per_w, a_per_w)])

    return gsum_k(table, idx2d)


def _sc_delta(table, a_msg, idx_rev2d, idx_b2a2d):
    D = table.shape[1]
    rows_per_w = NB_PAD // NW
    n_ch = rows_per_w // CH
    mesh = plsc.VectorSubcoreMesh(core_axis_name="c", subcore_axis_name="s")

    @functools.partial(
        pl.kernel,
        out_type=jax.ShapeDtypeStruct((NB_PAD, D), jnp.float32),
        mesh=mesh,
        compiler_params=pltpu.CompilerParams(use_tc_tiling_on_sc=False),
        scratch_types=[
            pltpu.VMEM((n_ch, CH), jnp.int32),
            pltpu.VMEM((n_ch, CH), jnp.int32),
            pltpu.VMEM((2 * CH, D), jnp.float32),
            pltpu.VMEM((2 * CH, D), jnp.float32),
            pltpu.VMEM((2 * CH, D), jnp.float32),
            pltpu.VMEM((2 * CH, D), jnp.float32),
            pltpu.VMEM_SHARED((NA_PAD, D), jnp.float32),
            pltpu.SemaphoreType.DMA,
            pltpu.SemaphoreType.DMA,
            pltpu.SemaphoreType.DMA,
            pltpu.SemaphoreType.DMA,
            pltpu.SemaphoreType.DMA,
            pltpu.SemaphoreType.DMA,
        ],
    )
    def delta_k(table_hbm, amsg_hbm, rev_hbm, b2a_hbm, out_hbm,
                irev_v, ib2a_v, rb0, rb1, ab0, ab1,
                shared, sr0, sr1, sa0, sa1, so0, so1):
        K = 2
        n_sch = n_ch // K
        wid = lax.axis_index("s") * NC + lax.axis_index("c")
        base = wid * rows_per_w

        @pl.when(lax.axis_index("s") == 0)
        def _():
            pltpu.sync_copy(amsg_hbm, shared)

        pltpu.sync_copy(rev_hbm.at[pl.ds(wid * n_ch, n_ch)], irev_v)
        pltpu.sync_copy(b2a_hbm.at[pl.ds(wid * n_ch, n_ch)], ib2a_v)
        plsc.subcore_barrier()

        def _fire(s, rb, ab, sr, sa):
            for b in range(K):
                pltpu.async_copy(table_hbm.at[irev_v.at[s * K + b]],
                                 rb.at[pl.ds(b * CH, CH)], sr)
                pltpu.async_copy(shared.at[ib2a_v.at[s * K + b]],
                                 ab.at[pl.ds(b * CH, CH)], sa)

        def _drain(rb, ab, sr, sa):
            pltpu.make_async_copy(table_hbm.at[pl.ds(0, K * CH)], rb, sr).wait()
            pltpu.make_async_copy(table_hbm.at[pl.ds(0, K * CH)], ab, sa).wait()

        def _emit(s, rb, ab, so):
            def one_chunk(c, carry):
                for rr in range(CH):
                    for k in range(_HV):
                        ab[c * CH + rr, pl.ds(k * 16, 16)] = (
                            ab[c * CH + rr, pl.ds(k * 16, 16)]
                            - rb[c * CH + rr, pl.ds(k * 16, 16)])
                return carry
            lax.fori_loop(0, K, one_chunk, 0)
            pltpu.async_copy(ab, out_hbm.at[pl.ds(base + s * K * CH, K * CH)], so)

        def _drain_out(ab, so):
            pltpu.make_async_copy(table_hbm.at[pl.ds(0, K * CH)], ab, so).wait()

        _fire(0, rb0, ab0, sr0, sa0)
        _fire(1, rb1, ab1, sr1, sa1)

        def outer(g, carry):
            s0 = g * 2
            _drain(rb0, ab0, sr0, sa0)
            _emit(s0, rb0, ab0, so0)
            _drain(rb1, ab1, sr1, sa1)
            _emit(s0 + 1, rb1, ab1, so1)

            @pl.when(s0 + 2 < n_sch)
            def _():
                _drain_out(ab0, so0)
                _fire(s0 + 2, rb0, ab0, sr0, sa0)

            @pl.when(s0 + 3 < n_sch)
            def _():
                _drain_out(ab1, so1)
                _fire(s0 + 3, rb1, ab1, sr1, sa1)
            return carry

        lax.fori_loop(0, n_sch // 2, outer, 0)
        _drain_out(ab0, so0)
        _drain_out(ab1, so1)

    return delta_k(table, a_msg, idx_rev2d, idx_b2a2d)


def _mm_in(f_bonds, w1b_t, bg1, wg2_t, bg2):
    blk = 2560
    grid = N_BONDS // blk

    def body(fb_ref, w_ref, b1_ref, w2_ref, b2_ref, fbp_ref, msg_ref):
        fbp = jnp.dot(fb_ref[...], w_ref[...], preferred_element_type=jnp.float32)
        fbp = fbp + b1_ref[...]
        fbp_ref[...] = fbp
        h = jnp.maximum(fbp, 0.0)
        m = jnp.dot(h, w2_ref[...], preferred_element_type=jnp.float32) + b2_ref[...]
        rows = lax.broadcasted_iota(jnp.int32, m.shape, 0)
        m = jnp.where(jnp.logical_and(rows == 0, pl.program_id(0) == 0), 0.0, m)
        msg_ref[...] = m

    return pl.pallas_call(
        body,
        grid=(grid,),
        in_specs=[
            pl.BlockSpec((blk, BOND_FDIM), lambda i: (i, 0)),
            pl.BlockSpec((BOND_FDIM, HIDDEN), lambda i: (0, 0)),
            pl.BlockSpec((1, HIDDEN), lambda i: (0, 0)),
            pl.BlockSpec((HIDDEN, HIDDEN), lambda i: (0, 0)),
            pl.BlockSpec((1, HIDDEN), lambda i: (0, 0)),
        ],
        out_specs=[
            pl.BlockSpec((blk, HIDDEN), lambda i: (i, 0)),
            pl.BlockSpec((blk, HIDDEN), lambda i: (i, 0)),
        ],
        out_shape=[
            jax.ShapeDtypeStruct((NB_PAD, HIDDEN), jnp.float32),
            jax.ShapeDtypeStruct((NB_PAD, HIDDEN), jnp.float32),
        ],
    )(f_bonds, w1b_t, bg1, wg2_t, bg2)


def _depth_update(fbp, delta, wmh_t, wg2_t, bg2):
    grid = NB_PAD // BOND_BLK

    def body(fbp_ref, d_ref, wm_ref, w2_ref, b2_ref, out_ref):
        h = fbp_ref[...] + jnp.dot(d_ref[...], wm_ref[...], preferred_element_type=jnp.float32)
        h = jnp.maximum(h, 0.0)
        m = jnp.dot(h, w2_ref[...], preferred_element_type=jnp.float32) + b2_ref[...]
        rows = lax.broadcasted_iota(jnp.int32, m.shape, 0)
        m = jnp.where(jnp.logical_and(rows == 0, pl.program_id(0) == 0), 0.0, m)
        out_ref[...] = m

    return pl.pallas_call(
        body,
        grid=(grid,),
        in_specs=[
            pl.BlockSpec((BOND_BLK, HIDDEN), lambda i: (i, 0)),
            pl.BlockSpec((BOND_BLK, HIDDEN), lambda i: (i, 0)),
            pl.BlockSpec((HIDDEN, HIDDEN), lambda i: (0, 0)),
            pl.BlockSpec((HIDDEN, HIDDEN), lambda i: (0, 0)),
            pl.BlockSpec((1, HIDDEN), lambda i: (0, 0)),
        ],
        out_specs=pl.BlockSpec((BOND_BLK, HIDDEN), lambda i: (i, 0)),
        out_shape=jax.ShapeDtypeStruct((NB_PAD, HIDDEN), jnp.float32),
    )(fbp, delta, wmh_t, wg2_t, bg2)


def _final_mlp(msgs, wm1_t, bm1, wm2_t, bm2):
    grid = NB_PAD // BOND_BLK
    H2 = 2 * HIDDEN

    def body(m0, m1, m2, m3, w1_ref, b1_ref, w2_ref, b2_ref, out_ref):
        s = jnp.dot(m0[...], w1_ref[0 * HIDDEN:1 * HIDDEN, :], preferred_element_type=jnp.float32)
        s += jnp.dot(m1[...], w1_ref[1 * HIDDEN:2 * HIDDEN, :], preferred_element_type=jnp.float32)
        s += jnp.dot(m2[...], w1_ref[2 * HIDDEN:3 * HIDDEN, :], preferred_element_type=jnp.float32)
        s += jnp.dot(m3[...], w1_ref[3 * HIDDEN:4 * HIDDEN, :], preferred_element_type=jnp.float32)
        h = jnp.maximum(s + b1_ref[...], 0.0)
        out_ref[...] = jnp.dot(h, w2_ref[...], preferred_element_type=jnp.float32) + b2_ref[...]

    mspec = pl.BlockSpec((BOND_BLK, HIDDEN), lambda i: (i, 0))
    return pl.pallas_call(
        body,
        grid=(grid,),
        in_specs=[
            mspec, mspec, mspec, mspec,
            pl.BlockSpec((DEPTH * HIDDEN, H2), lambda i: (0, 0)),
            pl.BlockSpec((1, H2), lambda i: (0, 0)),
            pl.BlockSpec((H2, HIDDEN), lambda i: (0, 0)),
            pl.BlockSpec((1, HIDDEN), lambda i: (0, 0)),
        ],
        out_specs=pl.BlockSpec((BOND_BLK, HIDDEN), lambda i: (i, 0)),
        out_shape=jax.ShapeDtypeStruct((NB_PAD, HIDDEN), jnp.float32),
    )(*msgs, wm1_t, bm1, wm2_t, bm2)


def _out_layer(a_sum, fa_pad, woa_t, wom_t, bo):
    grid = NA_PAD // ATOM_BLK

    def body(g_ref, fa_ref, wa_ref, wm_ref, b_ref, out_ref):
        x = jnp.dot(fa_ref[...], wa_ref[...], preferred_element_type=jnp.float32)
        x += jnp.dot(g_ref[...], wm_ref[...], preferred_element_type=jnp.float32)
        out_ref[...] = jnp.maximum(x + b_ref[...], 0.0)

    return pl.pallas_call(
        body,
        grid=(grid,),
        in_specs=[
            pl.BlockSpec((ATOM_BLK, HIDDEN), lambda i: (i, 0)),
            pl.BlockSpec((ATOM_BLK, ATOM_FDIM), lambda i: (i, 0)),
            pl.BlockSpec((ATOM_FDIM, HIDDEN), lambda i: (0, 0)),
            pl.BlockSpec((HIDDEN, HIDDEN), lambda i: (0, 0)),
            pl.BlockSpec((1, HIDDEN), lambda i: (0, 0)),
        ],
        out_specs=pl.BlockSpec((ATOM_BLK, HIDDEN), lambda i: (i, 0)),
        out_shape=jax.ShapeDtypeStruct((NA_PAD, HIDDEN), jnp.float32),
    )(a_sum, fa_pad, woa_t, wom_t, bo)


def kernel(f_atoms, f_bonds, a2b, b2a, b2revb, undirected_b2a,
           W_g1, b_g1, W_g2, b_g2, W_m1, b_m1, W_m2, b_m2, W_o, b_o):
    del undirected_b2a
    w1b_t = W_g1[:, :BOND_FDIM].T
    wmh_t = W_g1[:, BOND_FDIM:].T
    wg2_t = W_g2.T
    wm1_t = W_m1.T
    wm2_t = W_m2.T
    woa_t = W_o[:, :ATOM_FDIM].T
    wom_t = W_o[:, ATOM_FDIM:].T
    bg1 = b_g1[None, :]
    bg2 = b_g2[None, :]
    bm1 = b_m1[None, :]
    bm2 = b_m2[None, :]
    bo = b_o[None, :]

    a2b_p = jnp.pad(a2b, ((0, NA_PAD - N_ATOMS), (0, 0)))
    a2b2d = a2b_p.reshape(-1, CH)
    rev2d = jnp.pad(b2revb, (0, NB_PAD - N_BONDS)).reshape(-1, CH)
    b2a2d = jnp.pad(b2a, (0, NB_PAD - N_BONDS)).reshape(-1, CH)
    fa_pad = jnp.pad(f_atoms, ((0, NA_PAD - N_ATOMS), (0, 0)))

    fbp, msg = _mm_in(f_bonds, w1b_t, bg1, wg2_t, bg2)
    msgs = [msg]
    for _ in range(DEPTH - 1):
        a_msg = _sc_gather_sum(msg, a2b2d)
        delta = _sc_delta(msg, a_msg, rev2d, b2a2d)
        msg = _depth_update(fbp, delta, wmh_t, wg2_t, bg2)
        msgs.append(msg)

    tmp = _final_mlp(msgs, wm1_t, bm1, wm2_t, bm2)
    a_sum = _sc_gather_sum(tmp, a2b2d)
    out_pad = _out_layer(a_sum, fa_pad, woa_t, wom_t, bo)
    return out_pad[:N_ATOMS]


# speedup vs baseline: 1.1664x; 1.1664x over previous
import functools

import jax
import jax.numpy as jnp
from jax import lax
from jax.experimental import pallas as pl
from jax.experimental.pallas import tpu as pltpu
from jax.experimental.pallas import tpu_sc as plsc

DEPTH = 4
N_ATOMS = 10000
N_BONDS = 320000
MAX_NB = 32
ATOM_FDIM = 128
BOND_FDIM = 144
HIDDEN = 64

NC, NS = 2, 16
NW = NC * NS
CH = 128
NB_PAD = 327680
NA_PAD = 10240
BOND_BLK = 4096
ATOM_BLK = 512


_A_PER_CH = CH // MAX_NB
_HV = HIDDEN // 16


def _sc_gather_sum(table, idx2d):
    D = table.shape[1]
    rows_per_w = NB_PAD // NW
    n_ch = rows_per_w // CH
    a_per_w = NA_PAD // NW
    K = 4
    n_sch = n_ch // K
    mesh = plsc.VectorSubcoreMesh(core_axis_name="c", subcore_axis_name="s")

    @functools.partial(
        pl.kernel,
        out_type=jax.ShapeDtypeStruct((NA_PAD, D), jnp.bfloat16),
        mesh=mesh,
        compiler_params=pltpu.CompilerParams(use_tc_tiling_on_sc=False, needs_layout_passes=False),
        scratch_types=[
            pltpu.VMEM((n_ch, CH), jnp.int32),
            pltpu.VMEM((4 * CH, D), jnp.bfloat16),
            pltpu.VMEM((4 * CH, D), jnp.bfloat16),
            pltpu.VMEM((a_per_w, D), jnp.bfloat16),
            pltpu.SemaphoreType.DMA,
            pltpu.SemaphoreType.DMA,
        ],
    )
    def gsum_k(table_hbm, idx_hbm, out_hbm, idx_v, buf0, buf1, acc_v, sem0, sem1):
        wid = lax.axis_index("s") * NC + lax.axis_index("c")
        pltpu.sync_copy(idx_hbm.at[pl.ds(wid * n_ch, n_ch)], idx_v)

        def _fire(s, buf, sem):
            for b in range(K):
                pltpu.async_copy(table_hbm.at[idx_v.at[s * K + b]],
                                 buf.at[pl.ds(b * CH, CH)], sem)

        def _drain(buf, sem):
            pltpu.make_async_copy(table_hbm.at[pl.ds(0, K * CH)], buf, sem).wait()

        def _reduce(s, buf):
            def one_chunk(c, carry):
                for a in range(_A_PER_CH):
                    r0 = a * MAX_NB
                    for k in range(D // 32):
                        ea, eb = plsc.unpack(buf[c * CH + r0, pl.ds(k * 32, 32)],
                                             format=plsc.PackFormat.INTERLEAVED)
                        for j in range(1, MAX_NB):
                            xa, xb = plsc.unpack(
                                buf[c * CH + r0 + j, pl.ds(k * 32, 32)],
                                format=plsc.PackFormat.INTERLEAVED)
                            ea = ea + xa
                            eb = eb + xb
                        acc_v[(s * K + c) * _A_PER_CH + a, pl.ds(k * 32, 32)] = (
                            plsc.pack(ea, eb, format=plsc.PackFormat.INTERLEAVED))
                return carry
            lax.fori_loop(0, K, one_chunk, 0)

        _fire(0, buf0, sem0)

        def outer(g, carry):
            s0 = g * 2
            _fire(s0 + 1, buf1, sem1)
            _drain(buf0, sem0)
            _reduce(s0, buf0)

            @pl.when(s0 + 2 < n_sch)
            def _():
                _fire(s0 + 2, buf0, sem0)

            _drain(buf1, sem1)
            _reduce(s0 + 1, buf1)
            return carry

        lax.fori_loop(0, n_sch // 2, outer, 0)
        pltpu.sync_copy(acc_v, out_hbm.at[pl.ds(wid * a_per_w, a_per_w)])

    return gsum_k(table, idx2d)


def _sc_delta(table, a_msg, idx_rev2d, idx_b2a2d):
    D = table.shape[1]
    rows_per_w = NB_PAD // NW
    n_ch = rows_per_w // CH
    mesh = plsc.VectorSubcoreMesh(core_axis_name="c", subcore_axis_name="s")

    @functools.partial(
        pl.kernel,
        out_type=jax.ShapeDtypeStruct((NB_PAD, D), jnp.bfloat16),
        mesh=mesh,
        compiler_params=pltpu.CompilerParams(use_tc_tiling_on_sc=False, needs_layout_passes=False),
        scratch_types=[
            pltpu.VMEM((n_ch, CH), jnp.int32),
            pltpu.VMEM((n_ch, CH), jnp.int32),
            pltpu.VMEM((2 * CH, D), jnp.bfloat16),
            pltpu.VMEM((2 * CH, D), jnp.bfloat16),
            pltpu.VMEM((2 * CH, D), jnp.bfloat16),
            pltpu.VMEM((2 * CH, D), jnp.bfloat16),
            pltpu.VMEM_SHARED((NA_PAD, D), jnp.bfloat16),
            pltpu.SemaphoreType.DMA,
            pltpu.SemaphoreType.DMA,
            pltpu.SemaphoreType.DMA,
            pltpu.SemaphoreType.DMA,
            pltpu.SemaphoreType.DMA,
            pltpu.SemaphoreType.DMA,
        ],
    )
    def delta_k(table_hbm, amsg_hbm, rev_hbm, b2a_hbm, out_hbm,
                irev_v, ib2a_v, rb0, rb1, ab0, ab1,
                shared, sr0, sr1, sa0, sa1, so0, so1):
        K = 2
        n_sch = n_ch // K
        wid = lax.axis_index("s") * NC + lax.axis_index("c")
        base = wid * rows_per_w

        @pl.when(lax.axis_index("s") == 0)
        def _():
            pltpu.sync_copy(amsg_hbm, shared)

        pltpu.sync_copy(rev_hbm.at[pl.ds(wid * n_ch, n_ch)], irev_v)
        pltpu.sync_copy(b2a_hbm.at[pl.ds(wid * n_ch, n_ch)], ib2a_v)
        plsc.subcore_barrier()

        def _fire(s, rb, ab, sr, sa):
            for b in range(K):
                pltpu.async_copy(table_hbm.at[irev_v.at[s * K + b]],
                                 rb.at[pl.ds(b * CH, CH)], sr)
                pltpu.async_copy(shared.at[ib2a_v.at[s * K + b]],
                                 ab.at[pl.ds(b * CH, CH)], sa)

        def _drain(rb, ab, sr, sa):
            pltpu.make_async_copy(table_hbm.at[pl.ds(0, K * CH)], rb, sr).wait()
            pltpu.make_async_copy(table_hbm.at[pl.ds(0, K * CH)], ab, sa).wait()

        def _emit(s, rb, ab, so):
            def one_chunk(c, carry):
                for rr in range(CH):
                    for k in range(D // 32):
                        ab[c * CH + rr, pl.ds(k * 32, 32)] = (
                            ab[c * CH + rr, pl.ds(k * 32, 32)]
                            - rb[c * CH + rr, pl.ds(k * 32, 32)])
                return carry
            lax.fori_loop(0, K, one_chunk, 0)
            pltpu.async_copy(ab, out_hbm.at[pl.ds(base + s * K * CH, K * CH)], so)

        def _drain_out(ab, so):
            pltpu.make_async_copy(table_hbm.at[pl.ds(0, K * CH)], ab, so).wait()

        _fire(0, rb0, ab0, sr0, sa0)
        _fire(1, rb1, ab1, sr1, sa1)

        def outer(g, carry):
            s0 = g * 2
            _drain(rb0, ab0, sr0, sa0)
            _emit(s0, rb0, ab0, so0)
            _drain(rb1, ab1, sr1, sa1)
            _emit(s0 + 1, rb1, ab1, so1)

            @pl.when(s0 + 2 < n_sch)
            def _():
                _drain_out(ab0, so0)
                _fire(s0 + 2, rb0, ab0, sr0, sa0)

            @pl.when(s0 + 3 < n_sch)
            def _():
                _drain_out(ab1, so1)
                _fire(s0 + 3, rb1, ab1, sr1, sa1)
            return carry

        lax.fori_loop(0, n_sch // 2, outer, 0)
        _drain_out(ab0, so0)
        _drain_out(ab1, so1)

    return delta_k(table, a_msg, idx_rev2d, idx_b2a2d)


def _mm_in(f_bonds, w1b_t, bg1, wg2_t, bg2):
    blk = 2560
    grid = N_BONDS // blk

    def body(fb_ref, w_ref, b1_ref, w2_ref, b2_ref, fbp_ref, msg_ref):
        fbp = jnp.dot(fb_ref[...], w_ref[...], preferred_element_type=jnp.float32)
        fbp = fbp + b1_ref[...]
        fbp_ref[...] = fbp.astype(jnp.bfloat16)
        h = jnp.maximum(fbp, 0.0)
        m = jnp.dot(h, w2_ref[...], preferred_element_type=jnp.float32) + b2_ref[...]
        rows = lax.broadcasted_iota(jnp.int32, m.shape, 0)
        m = jnp.where(jnp.logical_and(rows == 0, pl.program_id(0) == 0), 0.0, m)
        msg_ref[...] = m.astype(jnp.bfloat16)

    return pl.pallas_call(
        body,
        grid=(grid,),
        in_specs=[
            pl.BlockSpec((blk, BOND_FDIM), lambda i: (i, 0)),
            pl.BlockSpec((BOND_FDIM, HIDDEN), lambda i: (0, 0)),
            pl.BlockSpec((1, HIDDEN), lambda i: (0, 0)),
            pl.BlockSpec((HIDDEN, HIDDEN), lambda i: (0, 0)),
            pl.BlockSpec((1, HIDDEN), lambda i: (0, 0)),
        ],
        out_specs=[
            pl.BlockSpec((blk, HIDDEN), lambda i: (i, 0)),
            pl.BlockSpec((blk, HIDDEN), lambda i: (i, 0)),
        ],
        out_shape=[
            jax.ShapeDtypeStruct((NB_PAD, HIDDEN), jnp.bfloat16),
            jax.ShapeDtypeStruct((NB_PAD, HIDDEN), jnp.bfloat16),
        ],
    )(f_bonds, w1b_t, bg1, wg2_t, bg2)


def _depth_update(fbp, delta, wmh_t, wg2_t, bg2):
    grid = NB_PAD // BOND_BLK

    def body(fbp_ref, d_ref, wm_ref, w2_ref, b2_ref, out_ref):
        d = d_ref[...].astype(jnp.float32)
        h = fbp_ref[...].astype(jnp.float32) + jnp.dot(
            d, wm_ref[...], preferred_element_type=jnp.float32)
        h = jnp.maximum(h, 0.0)
        m = jnp.dot(h, w2_ref[...], preferred_element_type=jnp.float32) + b2_ref[...]
        rows = lax.broadcasted_iota(jnp.int32, m.shape, 0)
        m = jnp.where(jnp.logical_and(rows == 0, pl.program_id(0) == 0), 0.0, m)
        out_ref[...] = m.astype(jnp.bfloat16)

    return pl.pallas_call(
        body,
        grid=(grid,),
        in_specs=[
            pl.BlockSpec((BOND_BLK, HIDDEN), lambda i: (i, 0)),
            pl.BlockSpec((BOND_BLK, HIDDEN), lambda i: (i, 0)),
            pl.BlockSpec((HIDDEN, HIDDEN), lambda i: (0, 0)),
            pl.BlockSpec((HIDDEN, HIDDEN), lambda i: (0, 0)),
            pl.BlockSpec((1, HIDDEN), lambda i: (0, 0)),
        ],
        out_specs=pl.BlockSpec((BOND_BLK, HIDDEN), lambda i: (i, 0)),
        out_shape=jax.ShapeDtypeStruct((NB_PAD, HIDDEN), jnp.bfloat16),
    )(fbp, delta, wmh_t, wg2_t, bg2)


def _final_mlp(msgs, wm1_t, bm1, wm2_t, bm2):
    grid = NB_PAD // BOND_BLK
    H2 = 2 * HIDDEN

    def body(m0, m1, m2, m3, w1_ref, b1_ref, w2_ref, b2_ref, out_ref):
        s = jnp.dot(m0[...].astype(jnp.float32), w1_ref[0 * HIDDEN:1 * HIDDEN, :], preferred_element_type=jnp.float32)
        s += jnp.dot(m1[...].astype(jnp.float32), w1_ref[1 * HIDDEN:2 * HIDDEN, :], preferred_element_type=jnp.float32)
        s += jnp.dot(m2[...].astype(jnp.float32), w1_ref[2 * HIDDEN:3 * HIDDEN, :], preferred_element_type=jnp.float32)
        s += jnp.dot(m3[...].astype(jnp.float32), w1_ref[3 * HIDDEN:4 * HIDDEN, :], preferred_element_type=jnp.float32)
        h = jnp.maximum(s + b1_ref[...], 0.0)
        t = jnp.dot(h, w2_ref[...], preferred_element_type=jnp.float32) + b2_ref[...]
        out_ref[...] = t.astype(jnp.bfloat16)

    mspec = pl.BlockSpec((BOND_BLK, HIDDEN), lambda i: (i, 0))
    return pl.pallas_call(
        body,
        grid=(grid,),
        in_specs=[
            mspec, mspec, mspec, mspec,
            pl.BlockSpec((DEPTH * HIDDEN, H2), lambda i: (0, 0)),
            pl.BlockSpec((1, H2), lambda i: (0, 0)),
            pl.BlockSpec((H2, HIDDEN), lambda i: (0, 0)),
            pl.BlockSpec((1, HIDDEN), lambda i: (0, 0)),
        ],
        out_specs=pl.BlockSpec((BOND_BLK, HIDDEN), lambda i: (i, 0)),
        out_shape=jax.ShapeDtypeStruct((NB_PAD, HIDDEN), jnp.bfloat16),
    )(*msgs, wm1_t, bm1, wm2_t, bm2)


def _out_layer(a_sum, fa_pad, woa_t, wom_t, bo):
    grid = NA_PAD // ATOM_BLK

    def body(g_ref, fa_ref, wa_ref, wm_ref, b_ref, out_ref):
        x = jnp.dot(fa_ref[...], wa_ref[...], preferred_element_type=jnp.float32)
        x += jnp.dot(g_ref[...].astype(jnp.float32), wm_ref[...],
                     preferred_element_type=jnp.float32)
        out_ref[...] = jnp.maximum(x + b_ref[...], 0.0)

    return pl.pallas_call(
        body,
        grid=(grid,),
        in_specs=[
            pl.BlockSpec((ATOM_BLK, HIDDEN), lambda i: (i, 0)),
            pl.BlockSpec((ATOM_BLK, ATOM_FDIM), lambda i: (i, 0)),
            pl.BlockSpec((ATOM_FDIM, HIDDEN), lambda i: (0, 0)),
            pl.BlockSpec((HIDDEN, HIDDEN), lambda i: (0, 0)),
            pl.BlockSpec((1, HIDDEN), lambda i: (0, 0)),
        ],
        out_specs=pl.BlockSpec((ATOM_BLK, HIDDEN), lambda i: (i, 0)),
        out_shape=jax.ShapeDtypeStruct((NA_PAD, HIDDEN), jnp.float32),
    )(a_sum, fa_pad, woa_t, wom_t, bo)


def kernel(f_atoms, f_bonds, a2b, b2a, b2revb, undirected_b2a,
           W_g1, b_g1, W_g2, b_g2, W_m1, b_m1, W_m2, b_m2, W_o, b_o):
    del undirected_b2a
    w1b_t = W_g1[:, :BOND_FDIM].T
    wmh_t = W_g1[:, BOND_FDIM:].T
    wg2_t = W_g2.T
    wm1_t = W_m1.T
    wm2_t = W_m2.T
    woa_t = W_o[:, :ATOM_FDIM].T
    wom_t = W_o[:, ATOM_FDIM:].T
    bg1 = b_g1[None, :]
    bg2 = b_g2[None, :]
    bm1 = b_m1[None, :]
    bm2 = b_m2[None, :]
    bo = b_o[None, :]

    a2b_p = jnp.pad(a2b, ((0, NA_PAD - N_ATOMS), (0, 0)))
    a2b2d = a2b_p.reshape(-1, CH)
    rev2d = jnp.pad(b2revb, (0, NB_PAD - N_BONDS)).reshape(-1, CH)
    b2a2d = jnp.pad(b2a, (0, NB_PAD - N_BONDS)).reshape(-1, CH)
    fa_pad = jnp.pad(f_atoms, ((0, NA_PAD - N_ATOMS), (0, 0)))

    fbp, msg = _mm_in(f_bonds, w1b_t, bg1, wg2_t, bg2)
    msgs = [msg]
    for _ in range(DEPTH - 1):
        a_msg = _sc_gather_sum(msg, a2b2d)
        delta = _sc_delta(msg, a_msg, rev2d, b2a2d)
        msg = _depth_update(fbp, delta, wmh_t, wg2_t, bg2)
        msgs.append(msg)

    tmp = _final_mlp(msgs, wm1_t, bm1, wm2_t, bm2)
    a_sum = _sc_gather_sum(tmp, a2b2d)
    out_pad = _out_layer(a_sum, fa_pad, woa_t, wom_t, bo)
    return out_pad[:N_ATOMS]


# speedup vs baseline: 1.2032x; 1.0316x over previous
import functools

import jax
import jax.numpy as jnp
from jax import lax
from jax.experimental import pallas as pl
from jax.experimental.pallas import tpu as pltpu
from jax.experimental.pallas import tpu_sc as plsc

DEPTH = 4
N_ATOMS = 10000
N_BONDS = 320000
MAX_NB = 32
ATOM_FDIM = 128
BOND_FDIM = 144
HIDDEN = 64

NC, NS = 2, 16
NW = NC * NS
CH = 128
NB_PAD = 327680
NA_PAD = 10240
BOND_BLK = 4096
ATOM_BLK = 512


_A_PER_CH = CH // MAX_NB
_HV = HIDDEN // 16


def _sc_gather_sum(table, idx2d):
    D = table.shape[1]
    rows_per_w = NB_PAD // NW
    n_ch = rows_per_w // CH
    a_per_w = NA_PAD // NW
    K = 4
    n_sch = n_ch // K
    mesh = plsc.VectorSubcoreMesh(core_axis_name="c", subcore_axis_name="s")

    @functools.partial(
        pl.kernel,
        out_type=jax.ShapeDtypeStruct((NA_PAD, D), jnp.bfloat16),
        mesh=mesh,
        compiler_params=pltpu.CompilerParams(use_tc_tiling_on_sc=False, needs_layout_passes=False),
        scratch_types=[
            pltpu.VMEM((n_ch, CH), jnp.int32),
            pltpu.VMEM((4 * CH, D), jnp.bfloat16),
            pltpu.VMEM((4 * CH, D), jnp.bfloat16),
            pltpu.VMEM((a_per_w, D), jnp.bfloat16),
            pltpu.SemaphoreType.DMA,
            pltpu.SemaphoreType.DMA,
        ],
    )
    def gsum_k(table_hbm, idx_hbm, out_hbm, idx_v, buf0, buf1, acc_v, sem0, sem1):
        wid = lax.axis_index("s") * NC + lax.axis_index("c")
        pltpu.sync_copy(idx_hbm.at[pl.ds(wid * n_ch, n_ch)], idx_v)

        def _fire(s, buf, sem):
            for b in range(K):
                pltpu.async_copy(table_hbm.at[idx_v.at[s * K + b]],
                                 buf.at[pl.ds(b * CH, CH)], sem)

        def _drain(buf, sem):
            pltpu.make_async_copy(table_hbm.at[pl.ds(0, K * CH)], buf, sem).wait()

        def _reduce(s, buf):
            def one_chunk(c, carry):
                for a in range(_A_PER_CH):
                    r0 = a * MAX_NB
                    for k in range(D // 32):
                        ea, eb = plsc.unpack(buf[c * CH + r0, pl.ds(k * 32, 32)],
                                             format=plsc.PackFormat.INTERLEAVED)
                        for j in range(1, MAX_NB):
                            xa, xb = plsc.unpack(
                                buf[c * CH + r0 + j, pl.ds(k * 32, 32)],
                                format=plsc.PackFormat.INTERLEAVED)
                            ea = ea + xa
                            eb = eb + xb
                        acc_v[(s * K + c) * _A_PER_CH + a, pl.ds(k * 32, 32)] = (
                            plsc.pack(ea, eb, format=plsc.PackFormat.INTERLEAVED))
                return carry
            lax.fori_loop(0, K, one_chunk, 0)

        _fire(0, buf0, sem0)

        def outer(g, carry):
            s0 = g * 2
            _fire(s0 + 1, buf1, sem1)
            _drain(buf0, sem0)
            _reduce(s0, buf0)

            @pl.when(s0 + 2 < n_sch)
            def _():
                _fire(s0 + 2, buf0, sem0)

            _drain(buf1, sem1)
            _reduce(s0 + 1, buf1)
            return carry

        lax.fori_loop(0, n_sch // 2, outer, 0)
        pltpu.sync_copy(acc_v, out_hbm.at[pl.ds(wid * a_per_w, a_per_w)])

    return gsum_k(table, idx2d)


def _sc_delta(table, a_msg, idx_rev2d, idx_b2a2d):
    D = table.shape[1]
    rows_per_w = NB_PAD // NW
    n_ch = rows_per_w // CH
    mesh = plsc.VectorSubcoreMesh(core_axis_name="c", subcore_axis_name="s")

    @functools.partial(
        pl.kernel,
        out_type=jax.ShapeDtypeStruct((NB_PAD, D), jnp.bfloat16),
        mesh=mesh,
        compiler_params=pltpu.CompilerParams(use_tc_tiling_on_sc=False, needs_layout_passes=False),
        scratch_types=[
            pltpu.VMEM((n_ch, CH), jnp.int32),
            pltpu.VMEM((n_ch, CH), jnp.int32),
            pltpu.VMEM((2 * CH, D), jnp.bfloat16),
            pltpu.VMEM((2 * CH, D), jnp.bfloat16),
            pltpu.VMEM((2 * CH, D), jnp.bfloat16),
            pltpu.VMEM((2 * CH, D), jnp.bfloat16),
            pltpu.VMEM_SHARED((NA_PAD, D), jnp.bfloat16),
            pltpu.SemaphoreType.DMA,
            pltpu.SemaphoreType.DMA,
            pltpu.SemaphoreType.DMA,
            pltpu.SemaphoreType.DMA,
            pltpu.SemaphoreType.DMA,
            pltpu.SemaphoreType.DMA,
        ],
    )
    def delta_k(table_hbm, amsg_hbm, rev_hbm, b2a_hbm, out_hbm,
                irev_v, ib2a_v, rb0, rb1, ab0, ab1,
                shared, sr0, sr1, sa0, sa1, so0, so1):
        K = 2
        n_sch = n_ch // K
        wid = lax.axis_index("s") * NC + lax.axis_index("c")
        base = wid * rows_per_w

        @pl.when(lax.axis_index("s") == 0)
        def _():
            pltpu.sync_copy(amsg_hbm, shared)

        pltpu.sync_copy(rev_hbm.at[pl.ds(wid * n_ch, n_ch)], irev_v)
        pltpu.sync_copy(b2a_hbm.at[pl.ds(wid * n_ch, n_ch)], ib2a_v)
        plsc.subcore_barrier()

        def _fire(s, rb, ab, sr, sa):
            for b in range(K):
                pltpu.async_copy(table_hbm.at[irev_v.at[s * K + b]],
                                 rb.at[pl.ds(b * CH, CH)], sr)
                pltpu.async_copy(shared.at[ib2a_v.at[s * K + b]],
                                 ab.at[pl.ds(b * CH, CH)], sa)

        def _drain(rb, ab, sr, sa):
            pltpu.make_async_copy(table_hbm.at[pl.ds(0, K * CH)], rb, sr).wait()
            pltpu.make_async_copy(table_hbm.at[pl.ds(0, K * CH)], ab, sa).wait()

        def _emit(s, rb, ab, so):
            def one_chunk(c, carry):
                for rr in range(CH):
                    for k in range(D // 32):
                        ab[c * CH + rr, pl.ds(k * 32, 32)] = (
                            ab[c * CH + rr, pl.ds(k * 32, 32)]
                            - rb[c * CH + rr, pl.ds(k * 32, 32)])
                return carry
            lax.fori_loop(0, K, one_chunk, 0)
            pltpu.async_copy(ab, out_hbm.at[pl.ds(base + s * K * CH, K * CH)], so)

        def _drain_out(ab, so):
            pltpu.make_async_copy(table_hbm.at[pl.ds(0, K * CH)], ab, so).wait()

        _fire(0, rb0, ab0, sr0, sa0)
        _fire(1, rb1, ab1, sr1, sa1)

        def outer(g, carry):
            s0 = g * 2
            _drain(rb0, ab0, sr0, sa0)
            _emit(s0, rb0, ab0, so0)
            _drain(rb1, ab1, sr1, sa1)
            _emit(s0 + 1, rb1, ab1, so1)

            @pl.when(s0 + 2 < n_sch)
            def _():
                _drain_out(ab0, so0)
                _fire(s0 + 2, rb0, ab0, sr0, sa0)

            @pl.when(s0 + 3 < n_sch)
            def _():
                _drain_out(ab1, so1)
                _fire(s0 + 3, rb1, ab1, sr1, sa1)
            return carry

        lax.fori_loop(0, n_sch // 2, outer, 0)
        _drain_out(ab0, so0)
        _drain_out(ab1, so1)

    return delta_k(table, a_msg, idx_rev2d, idx_b2a2d)


HB = NB_PAD // 2
PBLK = 2048


def _bd(w):
    z = jnp.zeros_like(w)
    return jnp.concatenate(
        [jnp.concatenate([w, z], axis=1), jnp.concatenate([z, w], axis=1)], axis=0)


def _mm_in(f_bonds2, w1b_2, bg1_2, wg2_2, bg2_2):
    blk = 1280
    grid = (N_BONDS // 2) // blk

    def body(fb_ref, w_ref, b1_ref, w2_ref, b2_ref, fbp_ref, msg_ref):
        fbp = jnp.dot(fb_ref[...], w_ref[...], preferred_element_type=jnp.float32)
        fbp = fbp + b1_ref[...]
        fbp_ref[...] = fbp.astype(jnp.bfloat16)
        h = jnp.maximum(fbp, 0.0)
        m = jnp.dot(h, w2_ref[...], preferred_element_type=jnp.float32) + b2_ref[...]
        rows = lax.broadcasted_iota(jnp.int32, m.shape, 0)
        cols = lax.broadcasted_iota(jnp.int32, m.shape, 1)
        m = jnp.where((rows == 0) & (cols < HIDDEN) & (pl.program_id(0) == 0), 0.0, m)
        msg_ref[...] = m.astype(jnp.bfloat16)

    return pl.pallas_call(
        body,
        grid=(grid,),
        in_specs=[
            pl.BlockSpec((blk, 2 * BOND_FDIM), lambda i: (i, 0)),
            pl.BlockSpec((2 * BOND_FDIM, 2 * HIDDEN), lambda i: (0, 0)),
            pl.BlockSpec((1, 2 * HIDDEN), lambda i: (0, 0)),
            pl.BlockSpec((2 * HIDDEN, 2 * HIDDEN), lambda i: (0, 0)),
            pl.BlockSpec((1, 2 * HIDDEN), lambda i: (0, 0)),
        ],
        out_specs=[
            pl.BlockSpec((blk, 2 * HIDDEN), lambda i: (i, 0)),
            pl.BlockSpec((blk, 2 * HIDDEN), lambda i: (i, 0)),
        ],
        out_shape=[
            jax.ShapeDtypeStruct((HB, 2 * HIDDEN), jnp.bfloat16),
            jax.ShapeDtypeStruct((HB, 2 * HIDDEN), jnp.bfloat16),
        ],
    )(f_bonds2, w1b_2, bg1_2, wg2_2, bg2_2)


def _depth_update(fbp2, delta2, wmh_2, wg2_2, bg2_2):
    grid = HB // PBLK

    def body(fbp_ref, d_ref, wm_ref, w2_ref, b2_ref, out_ref):
        d = d_ref[...].astype(jnp.float32)
        h = fbp_ref[...].astype(jnp.float32) + jnp.dot(
            d, wm_ref[...], preferred_element_type=jnp.float32)
        h = jnp.maximum(h, 0.0)
        m = jnp.dot(h, w2_ref[...], preferred_element_type=jnp.float32) + b2_ref[...]
        rows = lax.broadcasted_iota(jnp.int32, m.shape, 0)
        cols = lax.broadcasted_iota(jnp.int32, m.shape, 1)
        m = jnp.where((rows == 0) & (cols < HIDDEN) & (pl.program_id(0) == 0), 0.0, m)
        out_ref[...] = m.astype(jnp.bfloat16)

    return pl.pallas_call(
        body,
        grid=(grid,),
        in_specs=[
            pl.BlockSpec((PBLK, 2 * HIDDEN), lambda i: (i, 0)),
            pl.BlockSpec((PBLK, 2 * HIDDEN), lambda i: (i, 0)),
            pl.BlockSpec((2 * HIDDEN, 2 * HIDDEN), lambda i: (0, 0)),
            pl.BlockSpec((2 * HIDDEN, 2 * HIDDEN), lambda i: (0, 0)),
            pl.BlockSpec((1, 2 * HIDDEN), lambda i: (0, 0)),
        ],
        out_specs=pl.BlockSpec((PBLK, 2 * HIDDEN), lambda i: (i, 0)),
        out_shape=jax.ShapeDtypeStruct((HB, 2 * HIDDEN), jnp.bfloat16),
    )(fbp2, delta2, wmh_2, wg2_2, bg2_2)


def _final_mlp(msgs2, wm1_2, bm1_2, wm2_2, bm2_2):
    grid = HB // PBLK
    H2 = 2 * HIDDEN
    H4 = 4 * HIDDEN

    def body(m0, m1, m2, m3, w1_ref, b1_ref, w2_ref, b2_ref, out_ref):
        s = jnp.dot(m0[...].astype(jnp.float32), w1_ref[0 * H2:1 * H2, :], preferred_element_type=jnp.float32)
        s += jnp.dot(m1[...].astype(jnp.float32), w1_ref[1 * H2:2 * H2, :], preferred_element_type=jnp.float32)
        s += jnp.dot(m2[...].astype(jnp.float32), w1_ref[2 * H2:3 * H2, :], preferred_element_type=jnp.float32)
        s += jnp.dot(m3[...].astype(jnp.float32), w1_ref[3 * H2:4 * H2, :], preferred_element_type=jnp.float32)
        h = jnp.maximum(s + b1_ref[...], 0.0)
        t = jnp.dot(h, w2_ref[...], preferred_element_type=jnp.float32) + b2_ref[...]
        out_ref[...] = t.astype(jnp.bfloat16)

    mspec = pl.BlockSpec((PBLK, H2), lambda i: (i, 0))
    return pl.pallas_call(
        body,
        grid=(grid,),
        in_specs=[
            mspec, mspec, mspec, mspec,
            pl.BlockSpec((DEPTH * H2, H4), lambda i: (0, 0)),
            pl.BlockSpec((1, H4), lambda i: (0, 0)),
            pl.BlockSpec((H4, H2), lambda i: (0, 0)),
            pl.BlockSpec((1, H2), lambda i: (0, 0)),
        ],
        out_specs=pl.BlockSpec((PBLK, H2), lambda i: (i, 0)),
        out_shape=jax.ShapeDtypeStruct((HB, H2), jnp.bfloat16),
    )(*msgs2, wm1_2, bm1_2, wm2_2, bm2_2)


def _out_layer(a_sum, fa_pad, woa_t, wom_t, bo):
    grid = NA_PAD // ATOM_BLK

    def body(g_ref, fa_ref, wa_ref, wm_ref, b_ref, out_ref):
        x = jnp.dot(fa_ref[...], wa_ref[...], preferred_element_type=jnp.float32)
        x += jnp.dot(g_ref[...].astype(jnp.float32), wm_ref[...],
                     preferred_element_type=jnp.float32)
        out_ref[...] = jnp.maximum(x + b_ref[...], 0.0)

    return pl.pallas_call(
        body,
        grid=(grid,),
        in_specs=[
            pl.BlockSpec((ATOM_BLK, HIDDEN), lambda i: (i, 0)),
            pl.BlockSpec((ATOM_BLK, ATOM_FDIM), lambda i: (i, 0)),
            pl.BlockSpec((ATOM_FDIM, HIDDEN), lambda i: (0, 0)),
            pl.BlockSpec((HIDDEN, HIDDEN), lambda i: (0, 0)),
            pl.BlockSpec((1, HIDDEN), lambda i: (0, 0)),
        ],
        out_specs=pl.BlockSpec((ATOM_BLK, HIDDEN), lambda i: (i, 0)),
        out_shape=jax.ShapeDtypeStruct((NA_PAD, HIDDEN), jnp.float32),
    )(a_sum, fa_pad, woa_t, wom_t, bo)


def kernel(f_atoms, f_bonds, a2b, b2a, b2revb, undirected_b2a,
           W_g1, b_g1, W_g2, b_g2, W_m1, b_m1, W_m2, b_m2, W_o, b_o):
    del undirected_b2a
    w1b_2 = _bd(W_g1[:, :BOND_FDIM].T)
    wmh_2 = _bd(W_g1[:, BOND_FDIM:].T)
    wg2_2 = _bd(W_g2.T)
    wm1_t = W_m1.T
    wm1_2 = jnp.concatenate(
        [_bd(wm1_t[d * HIDDEN:(d + 1) * HIDDEN, :]) for d in range(DEPTH)], axis=0)
    wm2_2 = _bd(W_m2.T)
    woa_t = W_o[:, :ATOM_FDIM].T
    wom_t = W_o[:, ATOM_FDIM:].T
    bg1_2 = jnp.tile(b_g1[None, :], (1, 2))
    bg2_2 = jnp.tile(b_g2[None, :], (1, 2))
    bm1_2 = jnp.tile(b_m1[None, :], (1, 2))
    bm2_2 = jnp.tile(b_m2[None, :], (1, 2))
    bo = b_o[None, :]

    a2b_p = jnp.pad(a2b, ((0, NA_PAD - N_ATOMS), (0, 0)))
    a2b2d = a2b_p.reshape(-1, CH)
    rev2d = jnp.pad(b2revb, (0, NB_PAD - N_BONDS)).reshape(-1, CH)
    b2a2d = jnp.pad(b2a, (0, NB_PAD - N_BONDS)).reshape(-1, CH)
    fa_pad = jnp.pad(f_atoms, ((0, NA_PAD - N_ATOMS), (0, 0)))

    f_bonds2 = f_bonds.reshape(N_BONDS // 2, 2 * BOND_FDIM)
    fbp2, msg2 = _mm_in(f_bonds2, w1b_2, bg1_2, wg2_2, bg2_2)
    msgs2 = [msg2]
    for _ in range(DEPTH - 1):
        msg_flat = msg2.reshape(NB_PAD, HIDDEN)
        a_msg = _sc_gather_sum(msg_flat, a2b2d)
        delta = _sc_delta(msg_flat, a_msg, rev2d, b2a2d)
        msg2 = _depth_update(fbp2, delta.reshape(HB, 2 * HIDDEN),
                             wmh_2, wg2_2, bg2_2)
        msgs2.append(msg2)

    tmp2 = _final_mlp(msgs2, wm1_2, bm1_2, wm2_2, bm2_2)
    a_sum = _sc_gather_sum(tmp2.reshape(NB_PAD, HIDDEN), a2b2d)
    out_pad = _out_layer(a_sum, fa_pad, woa_t, wom_t, bo)
    return out_pad[:N_ATOMS]


# speedup vs baseline: 1.3643x; 1.1338x over previous
import functools

import jax
import jax.numpy as jnp
from jax import lax
from jax.experimental import pallas as pl
from jax.experimental.pallas import tpu as pltpu
from jax.experimental.pallas import tpu_sc as plsc

DEPTH = 4
N_ATOMS = 10000
N_BONDS = 320000
MAX_NB = 32
ATOM_FDIM = 128
BOND_FDIM = 144
HIDDEN = 64

NC, NS = 2, 16
NW = NC * NS
CH = 128
NB_PAD = 327680
NA_PAD = 10240
HB = NB_PAD // 2
PBLK = 2048

_A_PER_CH = CH // MAX_NB
_HV = HIDDEN // 16


def _sc_gather_sum(table2, idx2d):
    D = table2.shape[1] // 2
    rows_per_w = NB_PAD // NW
    n_ch = rows_per_w // CH
    a_per_w = NA_PAD // NW
    K = 4
    n_sch = n_ch // K
    mesh = plsc.VectorSubcoreMesh(core_axis_name="c", subcore_axis_name="s")

    @functools.partial(
        pl.kernel,
        out_type=jax.ShapeDtypeStruct((NA_PAD, D), jnp.float32),
        mesh=mesh,
        compiler_params=pltpu.CompilerParams(use_tc_tiling_on_sc=False,
                                             needs_layout_passes=False),
        scratch_types=[
            pltpu.VMEM((n_ch, CH), jnp.int32),
            pltpu.VMEM((K * CH, D), jnp.float32),
            pltpu.VMEM((K * CH, D), jnp.float32),
            pltpu.VMEM((a_per_w, D), jnp.float32),
            pltpu.SemaphoreType.DMA,
            pltpu.SemaphoreType.DMA,
        ],
    )
    def gsum_k(table_hbm, idx_hbm, out_hbm, idx_v, buf0, buf1, acc_v, sem0, sem1):
        wid = lax.axis_index("s") * NC + lax.axis_index("c")
        pltpu.sync_copy(idx_hbm.at[pl.ds(wid * n_ch, n_ch)], idx_v)

        def _fire(s, buf, sem):
            for b in range(K):
                pltpu.async_copy(table_hbm.at[idx_v.at[s * K + b]],
                                 buf.at[pl.ds(b * CH, CH)], sem)

        def _drain(buf, sem):
            pltpu.make_async_copy(table_hbm.at[pl.ds(0, K * CH)], buf, sem).wait()

        def _reduce(s, buf):
            def one_chunk(c, carry):
                for a in range(_A_PER_CH):
                    r0 = a * MAX_NB
                    for k in range(_HV):
                        acc = buf[c * CH + r0, pl.ds(k * 16, 16)]
                        for j in range(1, MAX_NB):
                            acc = acc + buf[c * CH + r0 + j, pl.ds(k * 16, 16)]
                        acc_v[(s * K + c) * _A_PER_CH + a, pl.ds(k * 16, 16)] = acc
                return carry
            lax.fori_loop(0, K, one_chunk, 0)

        _fire(0, buf0, sem0)

        def outer(g, carry):
            s0 = g * 2
            _fire(s0 + 1, buf1, sem1)
            _drain(buf0, sem0)
            _reduce(s0, buf0)

            @pl.when(s0 + 2 < n_sch)
            def _():
                _fire(s0 + 2, buf0, sem0)

            _drain(buf1, sem1)
            _reduce(s0 + 1, buf1)
            return carry

        lax.fori_loop(0, n_sch // 2, outer, 0)
        pltpu.sync_copy(acc_v, out_hbm.at[pl.ds(wid * a_per_w, a_per_w)])

    return gsum_k(table2.reshape(NB_PAD, D), idx2d)


def _sc_delta(table2, a_msg, idx_rev2d, idx_b2a2d):
    D = table2.shape[1] // 2
    rows_per_w = NB_PAD // NW
    n_ch = rows_per_w // CH
    mesh = plsc.VectorSubcoreMesh(core_axis_name="c", subcore_axis_name="s")

    @functools.partial(
        pl.kernel,
        out_type=jax.ShapeDtypeStruct((NB_PAD, D), jnp.float32),
        mesh=mesh,
        compiler_params=pltpu.CompilerParams(use_tc_tiling_on_sc=False,
                                             needs_layout_passes=False),
        scratch_types=[
            pltpu.VMEM((n_ch, CH), jnp.int32),
            pltpu.VMEM((n_ch, CH), jnp.int32),
            pltpu.VMEM((2 * CH, D), jnp.float32),
            pltpu.VMEM((2 * CH, D), jnp.float32),
            pltpu.VMEM((2 * CH, D), jnp.float32),
            pltpu.VMEM((2 * CH, D), jnp.float32),
            pltpu.VMEM_SHARED((NA_PAD, D), jnp.float32),
            pltpu.SemaphoreType.DMA,
            pltpu.SemaphoreType.DMA,
            pltpu.SemaphoreType.DMA,
            pltpu.SemaphoreType.DMA,
            pltpu.SemaphoreType.DMA,
            pltpu.SemaphoreType.DMA,
        ],
    )
    def delta_k(table_hbm, amsg_hbm, rev_hbm, b2a_hbm, out_hbm,
                irev_v, ib2a_v, rb0, rb1, ab0, ab1,
                shared, sr0, sr1, sa0, sa1, so0, so1):
        K = 2
        n_sch = n_ch // K
        wid = lax.axis_index("s") * NC + lax.axis_index("c")
        base = wid * rows_per_w

        @pl.when(lax.axis_index("s") == 0)
        def _():
            pltpu.sync_copy(amsg_hbm, shared)

        pltpu.sync_copy(rev_hbm.at[pl.ds(wid * n_ch, n_ch)], irev_v)
        pltpu.sync_copy(b2a_hbm.at[pl.ds(wid * n_ch, n_ch)], ib2a_v)
        plsc.subcore_barrier()

        def _fire(s, rb, ab, sr, sa):
            for b in range(K):
                pltpu.async_copy(table_hbm.at[irev_v.at[s * K + b]],
                                 rb.at[pl.ds(b * CH, CH)], sr)
                pltpu.async_copy(shared.at[ib2a_v.at[s * K + b]],
                                 ab.at[pl.ds(b * CH, CH)], sa)

        def _drain(rb, ab, sr, sa):
            pltpu.make_async_copy(table_hbm.at[pl.ds(0, K * CH)], rb, sr).wait()
            pltpu.make_async_copy(table_hbm.at[pl.ds(0, K * CH)], ab, sa).wait()

        def _emit(s, rb, ab, so):
            def one_chunk(c, carry):
                for rr in range(CH):
                    for k in range(_HV):
                        ab[c * CH + rr, pl.ds(k * 16, 16)] = (
                            ab[c * CH + rr, pl.ds(k * 16, 16)]
                            - rb[c * CH + rr, pl.ds(k * 16, 16)])
                return carry
            lax.fori_loop(0, K, one_chunk, 0)
            pltpu.async_copy(ab, out_hbm.at[pl.ds(base + s * K * CH, K * CH)], so)

        def _drain_out(ab, so):
            pltpu.make_async_copy(table_hbm.at[pl.ds(0, K * CH)], ab, so).wait()

        _fire(0, rb0, ab0, sr0, sa0)
        _fire(1, rb1, ab1, sr1, sa1)

        def outer(g, carry):
            s0 = g * 2
            _drain(rb0, ab0, sr0, sa0)
            _emit(s0, rb0, ab0, so0)
            _drain(rb1, ab1, sr1, sa1)
            _emit(s0 + 1, rb1, ab1, so1)

            @pl.when(s0 + 2 < n_sch)
            def _():
                _drain_out(ab0, so0)
                _fire(s0 + 2, rb0, ab0, sr0, sa0)

            @pl.when(s0 + 3 < n_sch)
            def _():
                _drain_out(ab1, so1)
                _fire(s0 + 3, rb1, ab1, sr1, sa1)
            return carry

        lax.fori_loop(0, n_sch // 2, outer, 0)
        _drain_out(ab0, so0)
        _drain_out(ab1, so1)

    return delta_k(table2.reshape(NB_PAD, D), a_msg, idx_rev2d, idx_b2a2d)


def _bd(w):
    z = jnp.zeros_like(w)
    return jnp.concatenate(
        [jnp.concatenate([w, z], axis=1), jnp.concatenate([z, w], axis=1)], axis=0)


def _mm_in(f_bonds2, w1b_2, bg1_2, wg2_2, bg2_2):
    blk = 1280
    grid = (N_BONDS // 2) // blk

    def body(fb_ref, w_ref, b1_ref, w2_ref, b2_ref, fbp_ref, msg_ref):
        fbp = jnp.dot(fb_ref[...], w_ref[...], preferred_element_type=jnp.float32)
        fbp = fbp + b1_ref[...]
        fbp_ref[...] = fbp
        h = jnp.maximum(fbp, 0.0)
        m = jnp.dot(h, w2_ref[...], preferred_element_type=jnp.float32) + b2_ref[...]
        rows = lax.broadcasted_iota(jnp.int32, m.shape, 0)
        cols = lax.broadcasted_iota(jnp.int32, m.shape, 1)
        m = jnp.where((rows == 0) & (cols < HIDDEN) & (pl.program_id(0) == 0), 0.0, m)
        msg_ref[...] = m

    return pl.pallas_call(
        body,
        grid=(grid,),
        in_specs=[
            pl.BlockSpec((blk, 2 * BOND_FDIM), lambda i: (i, 0)),
            pl.BlockSpec((2 * BOND_FDIM, 2 * HIDDEN), lambda i: (0, 0)),
            pl.BlockSpec((1, 2 * HIDDEN), lambda i: (0, 0)),
            pl.BlockSpec((2 * HIDDEN, 2 * HIDDEN), lambda i: (0, 0)),
            pl.BlockSpec((1, 2 * HIDDEN), lambda i: (0, 0)),
        ],
        out_specs=[
            pl.BlockSpec((blk, 2 * HIDDEN), lambda i: (i, 0)),
            pl.BlockSpec((blk, 2 * HIDDEN), lambda i: (i, 0)),
        ],
        out_shape=[
            jax.ShapeDtypeStruct((HB, 2 * HIDDEN), jnp.float32),
            jax.ShapeDtypeStruct((HB, 2 * HIDDEN), jnp.float32),
        ],
    )(f_bonds2, w1b_2, bg1_2, wg2_2, bg2_2)


def _depth_update(fbp2, delta2, wmh_2, wg2_2, bg2_2):
    grid = HB // PBLK

    def body(fbp_ref, d_ref, wm_ref, w2_ref, b2_ref, out_ref):
        h = fbp_ref[...] + jnp.dot(d_ref[...], wm_ref[...],
                                   preferred_element_type=jnp.float32)
        h = jnp.maximum(h, 0.0)
        m = jnp.dot(h, w2_ref[...], preferred_element_type=jnp.float32) + b2_ref[...]
        rows = lax.broadcasted_iota(jnp.int32, m.shape, 0)
        cols = lax.broadcasted_iota(jnp.int32, m.shape, 1)
        m = jnp.where((rows == 0) & (cols < HIDDEN) & (pl.program_id(0) == 0), 0.0, m)
        out_ref[...] = m

    return pl.pallas_call(
        body,
        grid=(grid,),
        in_specs=[
            pl.BlockSpec((PBLK, 2 * HIDDEN), lambda i: (i, 0)),
            pl.BlockSpec((PBLK, 2 * HIDDEN), lambda i: (i, 0)),
            pl.BlockSpec((2 * HIDDEN, 2 * HIDDEN), lambda i: (0, 0)),
            pl.BlockSpec((2 * HIDDEN, 2 * HIDDEN), lambda i: (0, 0)),
            pl.BlockSpec((1, 2 * HIDDEN), lambda i: (0, 0)),
        ],
        out_specs=pl.BlockSpec((PBLK, 2 * HIDDEN), lambda i: (i, 0)),
        out_shape=jax.ShapeDtypeStruct((HB, 2 * HIDDEN), jnp.float32),
    )(fbp2, delta2, wmh_2, wg2_2, bg2_2)


def _final_mlp(msgs2, wm1_2, bm1_2, wm2_2, bm2_2):
    grid = HB // PBLK
    H2 = 2 * HIDDEN
    H4 = 4 * HIDDEN

    def body(m0, m1, m2, m3, w1_ref, b1_ref, w2_ref, b2_ref, out_ref):
        s = jnp.dot(m0[...], w1_ref[0 * H2:1 * H2, :], preferred_element_type=jnp.float32)
        s += jnp.dot(m1[...], w1_ref[1 * H2:2 * H2, :], preferred_element_type=jnp.float32)
        s += jnp.dot(m2[...], w1_ref[2 * H2:3 * H2, :], preferred_element_type=jnp.float32)
        s += jnp.dot(m3[...], w1_ref[3 * H2:4 * H2, :], preferred_element_type=jnp.float32)
        h = jnp.maximum(s + b1_ref[...], 0.0)
        out_ref[...] = jnp.dot(h, w2_ref[...], preferred_element_type=jnp.float32) + b2_ref[...]

    mspec = pl.BlockSpec((PBLK, H2), lambda i: (i, 0))
    return pl.pallas_call(
        body,
        grid=(grid,),
        in_specs=[
            mspec, mspec, mspec, mspec,
            pl.BlockSpec((DEPTH * H2, H4), lambda i: (0, 0)),
            pl.BlockSpec((1, H4), lambda i: (0, 0)),
            pl.BlockSpec((H4, H2), lambda i: (0, 0)),
            pl.BlockSpec((1, H2), lambda i: (0, 0)),
        ],
        out_specs=pl.BlockSpec((PBLK, H2), lambda i: (i, 0)),
        out_shape=jax.ShapeDtypeStruct((HB, H2), jnp.float32),
    )(*msgs2, wm1_2, bm1_2, wm2_2, bm2_2)


def _out_layer(a_sum2, fa2, woa_2, wom_2, bo2):
    blk = 256
    grid = (NA_PAD // 2) // blk

    def body(g_ref, fa_ref, wa_ref, wm_ref, b_ref, out_ref):
        x = jnp.dot(fa_ref[...], wa_ref[...], preferred_element_type=jnp.float32)
        x += jnp.dot(g_ref[...], wm_ref[...], preferred_element_type=jnp.float32)
        out_ref[...] = jnp.maximum(x + b_ref[...], 0.0)

    return pl.pallas_call(
        body,
        grid=(grid,),
        in_specs=[
            pl.BlockSpec((blk, 2 * HIDDEN), lambda i: (i, 0)),
            pl.BlockSpec((blk, 2 * ATOM_FDIM), lambda i: (i, 0)),
            pl.BlockSpec((2 * ATOM_FDIM, 2 * HIDDEN), lambda i: (0, 0)),
            pl.BlockSpec((2 * HIDDEN, 2 * HIDDEN), lambda i: (0, 0)),
            pl.BlockSpec((1, 2 * HIDDEN), lambda i: (0, 0)),
        ],
        out_specs=pl.BlockSpec((blk, 2 * HIDDEN), lambda i: (i, 0)),
        out_shape=jax.ShapeDtypeStruct((NA_PAD // 2, 2 * HIDDEN), jnp.float32),
    )(a_sum2, fa2, woa_2, wom_2, bo2)


def kernel(f_atoms, f_bonds, a2b, b2a, b2revb, undirected_b2a,
           W_g1, b_g1, W_g2, b_g2, W_m1, b_m1, W_m2, b_m2, W_o, b_o):
    del undirected_b2a
    w1b_2 = _bd(W_g1[:, :BOND_FDIM].T)
    wmh_2 = _bd(W_g1[:, BOND_FDIM:].T)
    wg2_2 = _bd(W_g2.T)
    wm1_t = W_m1.T
    wm1_2 = jnp.concatenate(
        [_bd(wm1_t[d * HIDDEN:(d + 1) * HIDDEN, :]) for d in range(DEPTH)], axis=0)
    wm2_2 = _bd(W_m2.T)
    woa_2 = _bd(W_o[:, :ATOM_FDIM].T)
    wom_2 = _bd(W_o[:, ATOM_FDIM:].T)
    bg1_2 = jnp.tile(b_g1[None, :], (1, 2))
    bg2_2 = jnp.tile(b_g2[None, :], (1, 2))
    bm1_2 = jnp.tile(b_m1[None, :], (1, 2))
    bm2_2 = jnp.tile(b_m2[None, :], (1, 2))
    bo2 = jnp.tile(b_o[None, :], (1, 2))

    a2b_p = jnp.pad(a2b, ((0, NA_PAD - N_ATOMS), (0, 0)))
    a2b2d = a2b_p.reshape(-1, CH)
    rev2d = jnp.pad(b2revb, (0, NB_PAD - N_BONDS)).reshape(-1, CH)
    b2a2d = jnp.pad(b2a, (0, NB_PAD - N_BONDS)).reshape(-1, CH)
    fa_pad = jnp.pad(f_atoms, ((0, NA_PAD - N_ATOMS), (0, 0)))

    f_bonds2 = f_bonds.reshape(N_BONDS // 2, 2 * BOND_FDIM)
    fbp2, msg2 = _mm_in(f_bonds2, w1b_2, bg1_2, wg2_2, bg2_2)
    msgs2 = [msg2]
    for _ in range(DEPTH - 1):
        a_msg = _sc_gather_sum(msg2, a2b2d)
        delta = _sc_delta(msg2, a_msg, rev2d, b2a2d)
        msg2 = _depth_update(fbp2, delta.reshape(HB, 2 * HIDDEN), wmh_2, wg2_2, bg2_2)
        msgs2.append(msg2)

    tmp2 = _final_mlp(msgs2, wm1_2, bm1_2, wm2_2, bm2_2)
    a_sum = _sc_gather_sum(tmp2, a2b2d)
    fa2 = fa_pad.reshape(NA_PAD // 2, 2 * ATOM_FDIM)
    out2 = _out_layer(a_sum.reshape(NA_PAD // 2, 2 * HIDDEN), fa2, woa_2, wom_2, bo2)
    return out2.reshape(NA_PAD, HIDDEN)[:N_ATOMS]
